# Initial kernel scaffold; baseline (speedup 1.0000x reference)
import jax
import jax.numpy as jnp
from jax.experimental import pallas as pl


def kernel(user_table, item_table, group_table, adj_idx, adj_val, D, A, attn_w1, attn_b1, attn_w2, attn_b2, pred_w1, pred_b1, pred_w2, pred_b2, group_inputs, item_inputs, member_masked, mask):
    raise NotImplementedError("write your pallas kernel here")


# speedup vs baseline: 4.9366x; 4.9366x over previous
import functools

import jax
import jax.numpy as jnp
from jax import lax
from jax.experimental import pallas as pl
from jax.experimental.pallas import tpu as pltpu
from jax.experimental.pallas import tpu_sc as plsc

EMB = 32
NUSERS = 50000
NGROUPS = 2048
BSZ = 1024
MAXLEN = 10
NNZ = 1600000
PADJ = 1638400
EROWS = 12800
HALFP = 50048
NP = 2 * HALFP
TRASH = 50000
TILE_EROWS = 800
CHUNK_R = 4
CH = CHUNK_R * 128
NCHUNK = TILE_EROWS // CHUNK_R
RPT = HALFP // 16

_mesh = plsc.VectorSubcoreMesh(core_axis_name="c", subcore_axis_name="s")
_sc_params = pltpu.CompilerParams(use_tc_tiling_on_sc=False)

_f32 = jnp.float32
_i32 = jnp.int32


def _layer_body(col2d, rowf, valf, emb, out,
                col_v, rbuf, vbuf, lidx, gbuf, acc, sem_g, sem_s):
    c = lax.axis_index("c")
    s = lax.axis_index("s")

    def zb(e, _):
        gbuf[e, pl.ds(0, 16)] = jnp.zeros((16,), _f32)
        gbuf[e, pl.ds(16, 16)] = jnp.zeros((16,), _f32)
        return 0
    lax.fori_loop(0, CH, zb, 0, unroll=4)
    zbase = s * RPT
    for k in range(RPT // CH):
        pltpu.sync_copy(gbuf, acc.at[pl.ds(zbase + k * CH, CH)])
    _rem = RPT % CH
    pltpu.sync_copy(gbuf.at[pl.ds(0, _rem)],
                    acc.at[pl.ds(zbase + (RPT // CH) * CH, _rem)])
    plsc.subcore_barrier()

    base_row = c * HALFP
    tile_er0 = s * TILE_EROWS

    def chunk_body(ci, _):
        rb = tile_er0 + ci * CHUNK_R
        eb = rb * 128
        pltpu.sync_copy(col2d.at[pl.ds(rb, CHUNK_R)], col_v)
        pltpu.sync_copy(rowf.at[pl.ds(eb, CH)], rbuf)
        pltpu.sync_copy(valf.at[pl.ds(eb, CH)], vbuf)
        gd = [pltpu.async_copy(emb.at[col_v.at[r]],
                               gbuf.at[pl.ds(r * 128, 128)], sem_g)
              for r in range(CHUNK_R)]
        for d in gd:
            d.wait()
        for rr in range(CHUNK_R):
            def lx(k, _, rr=rr):
                rv = rbuf[pl.ds(rr * 128 + k * 16, 16)]
                lr = rv - base_row
                ok = (lr >= 0) & (lr < HALFP)
                lidx[rr, pl.ds(k * 16, 16)] = jnp.where(ok, lr, TRASH)
                return 0
            lax.fori_loop(0, 8, lx, 0, unroll=8)

        def mul16(m, _):
            vv = vbuf[pl.ds(m * 16, 16)]
            for j in range(16):
                e = m * 16 + j
                v = vv[j]
                gbuf[e, pl.ds(0, 16)] = gbuf[e, pl.ds(0, 16)] * v
                gbuf[e, pl.ds(16, 16)] = gbuf[e, pl.ds(16, 16)] * v
            return 0
        lax.fori_loop(0, CH // 16, mul16, 0)

        sd = [pltpu.async_copy(gbuf.at[pl.ds(r * 128, 128)],
                               acc.at[lidx.at[r]], sem_s, add=True)
              for r in range(CHUNK_R)]
        for d in sd:
            d.wait()
        return 0

    lax.fori_loop(0, NCHUNK, chunk_body, 0)
    plsc.subcore_barrier()

    obase = c * HALFP + s * RPT
    for k in range(RPT // CH):
        pltpu.sync_copy(acc.at[pl.ds(zbase + k * CH, CH)],
                        out.at[pl.ds(obase + k * CH, CH)])
    pltpu.sync_copy(acc.at[pl.ds(zbase + (RPT // CH) * CH, RPT % CH)],
                    out.at[pl.ds(obase + (RPT // CH) * CH, RPT % CH)])


_sc_layer = functools.partial(
    pl.kernel,
    out_type=jax.ShapeDtypeStruct((NP, EMB), _f32),
    mesh=_mesh,
    compiler_params=_sc_params,
    scratch_types=[
        pltpu.VMEM((CHUNK_R, 128), _i32),
        pltpu.VMEM((CH,), _i32),
        pltpu.VMEM((CH,), _f32),
        pltpu.VMEM((CHUNK_R, 128), _i32),
        pltpu.VMEM((CH, EMB), _f32),
        pltpu.VMEM_SHARED((HALFP, EMB), _f32),
        pltpu.SemaphoreType.DMA,
        pltpu.SemaphoreType.DMA,
    ],
)(_layer_body)


GIDX = 12288
GROWS = 96


def _gather4_body(t0, t1, t2, t3, idx2d, out,
                  idxv, b0, b1, b2, b3, sem):
    c = lax.axis_index("c")
    s = lax.axis_index("s")
    w = s * 2 + c
    for r in range(3):
        gr = w * 3 + r
        pltpu.sync_copy(idx2d.at[gr], idxv)
        gd = [pltpu.async_copy(tb.at[idxv], bb, sem)
              for tb, bb in ((t0, b0), (t1, b1), (t2, b2), (t3, b3))]
        for d in gd:
            d.wait()

        def sm(e, _):
            a = b0[e, pl.ds(0, 16)] + b1[e, pl.ds(0, 16)]
            bsum = b2[e, pl.ds(0, 16)] + b3[e, pl.ds(0, 16)]
            b0[e, pl.ds(0, 16)] = a + bsum
            a = b0[e, pl.ds(16, 16)] + b1[e, pl.ds(16, 16)]
            bsum = b2[e, pl.ds(16, 16)] + b3[e, pl.ds(16, 16)]
            b0[e, pl.ds(16, 16)] = a + bsum
            return 0
        lax.fori_loop(0, 128, sm, 0, unroll=4)
        pltpu.sync_copy(b0, out.at[pl.ds(gr * 128, 128)])


_sc_gather4 = functools.partial(
    pl.kernel,
    out_type=jax.ShapeDtypeStruct((GIDX, EMB), _f32),
    mesh=_mesh,
    compiler_params=_sc_params,
    scratch_types=[
        pltpu.VMEM((128,), _i32),
        pltpu.VMEM((128, EMB), _f32),
        pltpu.VMEM((128, EMB), _f32),
        pltpu.VMEM((128, EMB), _f32),
        pltpu.VMEM((128, EMB), _f32),
        pltpu.SemaphoreType.DMA,
    ],
)(_gather4_body)


def _tc_body(d_ref, a_ref, gt_ref, gi_ref, mem_ref, item_ref, mask_ref,
             w1u_ref, w1i_ref, b1_ref, w2_ref, b2_ref,
             pw1_ref, pb1_ref, pw2_ref, pb2_ref, out_ref):
    f32 = _f32

    def mm(x, y):
        return jax.lax.dot(x, y, preferred_element_type=f32)

    g0 = gt_ref[...]
    dmat = d_ref[...]
    amat = a_ref[...]
    g1 = mm(dmat, mm(amat, g0))
    g2 = mm(dmat, mm(amat, g1))
    g3 = mm(dmat, mm(amat, g2))
    gacc = g0 + g1 + g2 + g3

    cols = lax.broadcasted_iota(_i32, (BSZ, NGROUPS), 1)
    oh = (cols == gi_ref[...]).astype(f32)
    g_pure = mm(oh, gacc)

    item = item_ref[...]
    q = mm(item, w1i_ref[...]) + b1_ref[...]
    w1u = w1u_ref[...]
    w2 = w2_ref[...]
    b2v = b2_ref[...]
    scores = []
    for l in range(MAXLEN):
        m_l = mem_ref[pl.ds(l * BSZ, BSZ), :]
        h = jnp.maximum(mm(m_l, w1u) + q, 0.0)
        scores.append(mm(h, w2) + b2v)
    smat = jnp.concatenate(scores, axis=1)
    smat = jnp.where(mask_ref[...] != 0, jnp.float32(-1e30), smat)
    mx = jnp.max(smat, axis=1, keepdims=True)
    ex = jnp.exp(smat - mx)
    wt = ex / jnp.sum(ex, axis=1, keepdims=True)

    g_att = jnp.zeros((BSZ, EMB), f32)
    for l in range(MAXLEN):
        m_l = mem_ref[pl.ds(l * BSZ, BSZ), :]
        g_att = g_att + m_l * wt[:, l:l + 1]

    group_emb = g_att + g_pure
    elem = group_emb * item
    h2 = (mm(elem, pw1_ref[pl.ds(0, EMB), :])
          + mm(group_emb, pw1_ref[pl.ds(EMB, EMB), :])
          + mm(item, pw1_ref[pl.ds(2 * EMB, EMB), :])
          + pb1_ref[...])
    h2 = jnp.maximum(h2, 0.0)
    z = mm(h2, pw2_ref[...]) + pb2_ref[...]
    out_ref[...] = 1.0 / (1.0 + jnp.exp(-z))


def _tc_main(D, A, gt, gi, mem, item, mask,
             w1u, w1i, b1, w2, b2, pw1, pb1, pw2, pb2):
    return pl.pallas_call(
        _tc_body,
        out_shape=jax.ShapeDtypeStruct((BSZ, 1), _f32),
    )(D, A, gt, gi, mem, item, mask, w1u, w1i, b1, w2, b2,
      pw1, pb1, pw2, pb2)


def _remap(r):
    return r + jnp.where(r >= NUSERS, 48, 0).astype(_i32)


def kernel(user_table, item_table, group_table, adj_idx, adj_val, D, A,
           attn_w1, attn_b1, attn_w2, attn_b2,
           pred_w1, pred_b1, pred_w2, pred_b2,
           group_inputs, item_inputs, member_masked, mask):
    pad = PADJ - NNZ
    row_p = jnp.pad(_remap(adj_idx[0]), (0, pad))
    col_p = jnp.pad(_remap(adj_idx[1]), (0, pad)).reshape(EROWS, 128)
    val_p = jnp.pad(adj_val, (0, pad))

    zpad = jnp.zeros((48, EMB), _f32)
    ui_p = jnp.concatenate([user_table, zpad, item_table, zpad], axis=0)

    e1 = _sc_layer(col_p, row_p, val_p, ui_p)
    e2 = _sc_layer(col_p, row_p, val_p, e1)
    e3 = _sc_layer(col_p, row_p, val_p, e2)

    gidx = jnp.concatenate([
        member_masked.T.reshape(-1),
        item_inputs + HALFP,
        jnp.zeros((GIDX - BSZ * MAXLEN - BSZ,), _i32),
    ]).reshape(GROWS, 128)
    gsum = _sc_gather4(ui_p, e1, e2, e3, gidx)
    mem = gsum[:BSZ * MAXLEN]
    item_emb = gsum[BSZ * MAXLEN:BSZ * MAXLEN + BSZ]

    return _tc_main(
        D, A, group_table, group_inputs.reshape(BSZ, 1), mem, item_emb,
        mask, attn_w1[:EMB], attn_w1[EMB:], attn_b1.reshape(1, 1 * EMB),
        attn_w2, attn_b2.reshape(1, 1), pred_w1, pred_b1.reshape(1, 8),
        pred_w2, pred_b2.reshape(1, 1))


# speedup vs baseline: 5.2943x; 1.0724x over previous
import functools

import jax
import jax.numpy as jnp
from jax import lax
from jax.experimental import pallas as pl
from jax.experimental.pallas import tpu as pltpu
from jax.experimental.pallas import tpu_sc as plsc

EMB = 32
NUSERS = 50000
NGROUPS = 2048
BSZ = 1024
MAXLEN = 10
NNZ = 1600000
PADJ = 1638400
EROWS = 12800
HALFP = 50048
NP = 2 * HALFP
TRASH = 50000
TILE_EROWS = 800
CHUNK_R = 4
CH = CHUNK_R * 128
NCHUNK = TILE_EROWS // CHUNK_R
RPT = HALFP // 16

_mesh = plsc.VectorSubcoreMesh(core_axis_name="c", subcore_axis_name="s")
_sc_params = pltpu.CompilerParams(use_tc_tiling_on_sc=False)

_f32 = jnp.float32
_i32 = jnp.int32


def _layer_body(col2d, rowf, valf, emb, out,
                col_v, rbuf, vbuf, lidx, gbuf, acc, sem_g, sem_s):
    c = lax.axis_index("c")
    s = lax.axis_index("s")

    def zb(e, _):
        gbuf[e, pl.ds(0, 16)] = jnp.zeros((16,), _f32)
        gbuf[e, pl.ds(16, 16)] = jnp.zeros((16,), _f32)
        return 0
    lax.fori_loop(0, CH, zb, 0, unroll=4)
    zbase = s * RPT
    for k in range(RPT // CH):
        pltpu.sync_copy(gbuf, acc.at[pl.ds(zbase + k * CH, CH)])
    _rem = RPT % CH
    pltpu.sync_copy(gbuf.at[pl.ds(0, _rem)],
                    acc.at[pl.ds(zbase + (RPT // CH) * CH, _rem)])
    plsc.subcore_barrier()

    base_row = c * HALFP
    tile_er0 = s * TILE_EROWS

    def chunk_body(ci, _):
        rb = tile_er0 + ci * CHUNK_R
        eb = rb * 128
        pltpu.sync_copy(col2d.at[pl.ds(rb, CHUNK_R)], col_v)
        pltpu.sync_copy(rowf.at[pl.ds(eb, CH)], rbuf)
        pltpu.sync_copy(valf.at[pl.ds(eb, CH)], vbuf)
        gd = [pltpu.async_copy(emb.at[col_v.at[r]],
                               gbuf.at[pl.ds(r * 128, 128)], sem_g)
              for r in range(CHUNK_R)]
        for d in gd:
            d.wait()
        iota16 = lax.broadcasted_iota(_i32, (16,), 0)
        for rr in range(CHUNK_R):
            def lx(k, _, rr=rr):
                off = rr * 128 + k * 16
                rv = rbuf[pl.ds(off, 16)]
                lr = rv - base_row
                ok = (lr >= 0) & (lr < HALFP)
                spread = zbase + off + iota16
                lidx[rr, pl.ds(k * 16, 16)] = jnp.where(ok, lr, spread)
                vbuf[pl.ds(off, 16)] = jnp.where(
                    ok, vbuf[pl.ds(off, 16)], 0.0)
                return 0
            lax.fori_loop(0, 8, lx, 0, unroll=8)

        def mul16(m, _):
            vv = vbuf[pl.ds(m * 16, 16)]
            for j in range(16):
                e = m * 16 + j
                v = vv[j]
                gbuf[e, pl.ds(0, 16)] = gbuf[e, pl.ds(0, 16)] * v
                gbuf[e, pl.ds(16, 16)] = gbuf[e, pl.ds(16, 16)] * v
            return 0
        lax.fori_loop(0, CH // 16, mul16, 0)

        sd = [pltpu.async_copy(gbuf.at[pl.ds(r * 128, 128)],
                               acc.at[lidx.at[r]], sem_s, add=True)
              for r in range(CHUNK_R)]
        for d in sd:
            d.wait()
        return 0

    lax.fori_loop(0, NCHUNK, chunk_body, 0)
    plsc.subcore_barrier()

    obase = c * HALFP + s * RPT
    for k in range(RPT // CH):
        pltpu.sync_copy(acc.at[pl.ds(zbase + k * CH, CH)],
                        out.at[pl.ds(obase + k * CH, CH)])
    pltpu.sync_copy(acc.at[pl.ds(zbase + (RPT // CH) * CH, RPT % CH)],
                    out.at[pl.ds(obase + (RPT // CH) * CH, RPT % CH)])


_sc_layer = functools.partial(
    pl.kernel,
    out_type=jax.ShapeDtypeStruct((NP, EMB), _f32),
    mesh=_mesh,
    compiler_params=_sc_params,
    scratch_types=[
        pltpu.VMEM((CHUNK_R, 128), _i32),
        pltpu.VMEM((CH,), _i32),
        pltpu.VMEM((CH,), _f32),
        pltpu.VMEM((CHUNK_R, 128), _i32),
        pltpu.VMEM((CH, EMB), _f32),
        pltpu.VMEM_SHARED((HALFP, EMB), _f32),
        pltpu.SemaphoreType.DMA,
        pltpu.SemaphoreType.DMA,
    ],
)(_layer_body)


GIDX = 12288
GROWS = 96


def _gather4_body(t0, t1, t2, t3, idx2d, out,
                  idxv, b0, b1, b2, b3, sem):
    c = lax.axis_index("c")
    s = lax.axis_index("s")
    w = s * 2 + c
    for r in range(3):
        gr = w * 3 + r
        pltpu.sync_copy(idx2d.at[gr], idxv)
        gd = [pltpu.async_copy(tb.at[idxv], bb, sem)
              for tb, bb in ((t0, b0), (t1, b1), (t2, b2), (t3, b3))]
        for d in gd:
            d.wait()

        def sm(e, _):
            a = b0[e, pl.ds(0, 16)] + b1[e, pl.ds(0, 16)]
            bsum = b2[e, pl.ds(0, 16)] + b3[e, pl.ds(0, 16)]
            b0[e, pl.ds(0, 16)] = a + bsum
            a = b0[e, pl.ds(16, 16)] + b1[e, pl.ds(16, 16)]
            bsum = b2[e, pl.ds(16, 16)] + b3[e, pl.ds(16, 16)]
            b0[e, pl.ds(16, 16)] = a + bsum
            return 0
        lax.fori_loop(0, 128, sm, 0, unroll=4)
        pltpu.sync_copy(b0, out.at[pl.ds(gr * 128, 128)])


_sc_gather4 = functools.partial(
    pl.kernel,
    out_type=jax.ShapeDtypeStruct((GIDX, EMB), _f32),
    mesh=_mesh,
    compiler_params=_sc_params,
    scratch_types=[
        pltpu.VMEM((128,), _i32),
        pltpu.VMEM((128, EMB), _f32),
        pltpu.VMEM((128, EMB), _f32),
        pltpu.VMEM((128, EMB), _f32),
        pltpu.VMEM((128, EMB), _f32),
        pltpu.SemaphoreType.DMA,
    ],
)(_gather4_body)


def _tc_body(d_ref, a_ref, gt_ref, gi_ref, mem_ref, item_ref, mask_ref,
             w1u_ref, w1i_ref, b1_ref, w2_ref, b2_ref,
             pw1_ref, pb1_ref, pw2_ref, pb2_ref, out_ref):
    f32 = _f32

    def mm(x, y):
        return jax.lax.dot(x, y, preferred_element_type=f32)

    g0 = gt_ref[...]
    dmat = d_ref[...]
    amat = a_ref[...]
    g1 = mm(dmat, mm(amat, g0))
    g2 = mm(dmat, mm(amat, g1))
    g3 = mm(dmat, mm(amat, g2))
    gacc = g0 + g1 + g2 + g3

    cols = lax.broadcasted_iota(_i32, (BSZ, NGROUPS), 1)
    oh = (cols == gi_ref[...]).astype(f32)
    g_pure = mm(oh, gacc)

    item = item_ref[...]
    q = mm(item, w1i_ref[...]) + b1_ref[...]
    w1u = w1u_ref[...]
    w2 = w2_ref[...]
    b2v = b2_ref[...]
    scores = []
    for l in range(MAXLEN):
        m_l = mem_ref[pl.ds(l * BSZ, BSZ), :]
        h = jnp.maximum(mm(m_l, w1u) + q, 0.0)
        scores.append(mm(h, w2) + b2v)
    smat = jnp.concatenate(scores, axis=1)
    smat = jnp.where(mask_ref[...] != 0, jnp.float32(-1e30), smat)
    mx = jnp.max(smat, axis=1, keepdims=True)
    ex = jnp.exp(smat - mx)
    wt = ex / jnp.sum(ex, axis=1, keepdims=True)

    g_att = jnp.zeros((BSZ, EMB), f32)
    for l in range(MAXLEN):
        m_l = mem_ref[pl.ds(l * BSZ, BSZ), :]
        g_att = g_att + m_l * wt[:, l:l + 1]

    group_emb = g_att + g_pure
    elem = group_emb * item
    h2 = (mm(elem, pw1_ref[pl.ds(0, EMB), :])
          + mm(group_emb, pw1_ref[pl.ds(EMB, EMB), :])
          + mm(item, pw1_ref[pl.ds(2 * EMB, EMB), :])
          + pb1_ref[...])
    h2 = jnp.maximum(h2, 0.0)
    z = mm(h2, pw2_ref[...]) + pb2_ref[...]
    out_ref[...] = 1.0 / (1.0 + jnp.exp(-z))


def _tc_main(D, A, gt, gi, mem, item, mask,
             w1u, w1i, b1, w2, b2, pw1, pb1, pw2, pb2):
    return pl.pallas_call(
        _tc_body,
        out_shape=jax.ShapeDtypeStruct((BSZ, 1), _f32),
    )(D, A, gt, gi, mem, item, mask, w1u, w1i, b1, w2, b2,
      pw1, pb1, pw2, pb2)


def _remap(r):
    return r + jnp.where(r >= NUSERS, 48, 0).astype(_i32)


def kernel(user_table, item_table, group_table, adj_idx, adj_val, D, A,
           attn_w1, attn_b1, attn_w2, attn_b2,
           pred_w1, pred_b1, pred_w2, pred_b2,
           group_inputs, item_inputs, member_masked, mask):
    pad = PADJ - NNZ
    row_p = jnp.pad(_remap(adj_idx[0]), (0, pad))
    col_p = jnp.pad(_remap(adj_idx[1]), (0, pad)).reshape(EROWS, 128)
    val_p = jnp.pad(adj_val, (0, pad))

    zpad = jnp.zeros((48, EMB), _f32)
    ui_p = jnp.concatenate([user_table, zpad, item_table, zpad], axis=0)

    e1 = _sc_layer(col_p, row_p, val_p, ui_p)
    e2 = _sc_layer(col_p, row_p, val_p, e1)
    e3 = _sc_layer(col_p, row_p, val_p, e2)

    gidx = jnp.concatenate([
        member_masked.T.reshape(-1),
        item_inputs + HALFP,
        jnp.zeros((GIDX - BSZ * MAXLEN - BSZ,), _i32),
    ]).reshape(GROWS, 128)
    gsum = _sc_gather4(ui_p, e1, e2, e3, gidx)
    mem = gsum[:BSZ * MAXLEN]
    item_emb = gsum[BSZ * MAXLEN:BSZ * MAXLEN + BSZ]

    return _tc_main(
        D, A, group_table, group_inputs.reshape(BSZ, 1), mem, item_emb,
        mask, attn_w1[:EMB], attn_w1[EMB:], attn_b1.reshape(1, 1 * EMB),
        attn_w2, attn_b2.reshape(1, 1), pred_w1, pred_b1.reshape(1, 8),
        pred_w2, pred_b2.reshape(1, 1))


# speedup vs baseline: 6.6765x; 1.2611x over previous
import functools

import jax
import jax.numpy as jnp
from jax import lax
from jax.experimental import pallas as pl
from jax.experimental.pallas import tpu as pltpu
from jax.experimental.pallas import tpu_sc as plsc

EMB = 32
NUSERS = 50000
NGROUPS = 2048
BSZ = 1024
MAXLEN = 10
NNZ = 1600000
PADJ = 1638400
EROWS = 12800
HALFP = 50048
NP = 2 * HALFP
TILE_EROWS = 800
BLK_R = 8
BLK_E = BLK_R * 128
NBLK = TILE_EROWS // BLK_R
NSUB = BLK_R
RING = 4
RPT = HALFP // 16

_mesh = plsc.VectorSubcoreMesh(core_axis_name="c", subcore_axis_name="s")
_sc_params = pltpu.CompilerParams(use_tc_tiling_on_sc=False)

_f32 = jnp.float32
_i32 = jnp.int32


def _layer_body(col2d, rowf, valf, emb, out,
                sc0, sr0, sv0, sc1, sr1, sv1, lidx, gb, acc,
                si0, si1, sg0, sg1, sg2, sg3, ss0, ss1, ss2, ss3):
    c = lax.axis_index("c")
    s = lax.axis_index("s")
    GBR = RING * 128

    def zb(e, _):
        gb[e, pl.ds(0, 16)] = jnp.zeros((16,), _f32)
        gb[e, pl.ds(16, 16)] = jnp.zeros((16,), _f32)
        return 0
    lax.fori_loop(0, GBR, zb, 0, unroll=4)
    zbase = s * RPT
    for k in range(RPT // GBR):
        pltpu.sync_copy(gb, acc.at[pl.ds(zbase + k * GBR, GBR)])
    _rem = RPT % GBR
    pltpu.sync_copy(gb.at[pl.ds(0, _rem)],
                    acc.at[pl.ds(zbase + (RPT // GBR) * GBR, _rem)])
    plsc.subcore_barrier()

    base_row = c * HALFP
    tile_er0 = s * TILE_EROWS
    iota16 = lax.broadcasted_iota(_i32, (16,), 0)
    sg = (sg0, sg1, sg2, sg3)
    ss = (ss0, ss1, ss2, ss3)
    stages = ((sc0, sr0, sv0), (sc1, sr1, sv1))

    def body2(g, _):
        rb0 = tile_er0 + g * 2 * BLK_R
        eb0 = rb0 * 128
        dst0 = [pltpu.async_copy(col2d.at[pl.ds(rb0, BLK_R)], sc0, si0),
                pltpu.async_copy(rowf.at[pl.ds(eb0, BLK_E)], sr0, si0),
                pltpu.async_copy(valf.at[pl.ds(eb0, BLK_E)], sv0, si0)]
        dst1 = [pltpu.async_copy(
                    col2d.at[pl.ds(rb0 + BLK_R, BLK_R)], sc1, si1),
                pltpu.async_copy(
                    rowf.at[pl.ds(eb0 + BLK_E, BLK_E)], sr1, si1),
                pltpu.async_copy(
                    valf.at[pl.ds(eb0 + BLK_E, BLK_E)], sv1, si1)]
        for d in dst0:
            d.wait()

        def issue_gather(u):
            pb, sub = divmod(u, NSUB)
            q = u % RING
            return pltpu.async_copy(emb.at[stages[pb][0].at[sub]],
                                    gb.at[pl.ds(q * 128, 128)], sg[q])

        dsg = [None] * RING
        dss = [None] * RING
        dsg[0] = issue_gather(0)
        dsg[1] = issue_gather(1)
        for u in range(2 * NSUB):
            pb, sub = divmod(u, NSUB)
            q = u % RING
            un = u + 2
            if un < 2 * NSUB:
                if un == NSUB:
                    for d in dst1:
                        d.wait()
                qn = un % RING
                if dss[qn] is not None:
                    dss[qn].wait()
                    dss[qn] = None
                dsg[qn] = issue_gather(un)
            dsg[q].wait()
            _, strow, stval = stages[pb]
            vo = sub * 128
            for k in range(8):
                off = vo + k * 16
                rv = strow[pl.ds(off, 16)]
                lr = rv - base_row
                ok = (lr >= 0) & (lr < HALFP)
                spread = zbase + off + iota16
                lidx[q, pl.ds(k * 16, 16)] = jnp.where(ok, lr, spread)
                stval[pl.ds(off, 16)] = jnp.where(
                    ok, stval[pl.ds(off, 16)], 0.0)

            def mul16(m, _, q=q, stval=stval, vo=vo):
                vv = stval[pl.ds(vo + m * 16, 16)]
                for j in range(16):
                    e = q * 128 + m * 16 + j
                    v = vv[j]
                    gb[e, pl.ds(0, 16)] = gb[e, pl.ds(0, 16)] * v
                    gb[e, pl.ds(16, 16)] = gb[e, pl.ds(16, 16)] * v
                return 0
            lax.fori_loop(0, 8, mul16, 0)

            dss[q] = pltpu.async_copy(gb.at[pl.ds(q * 128, 128)],
                                      acc.at[lidx.at[q]], ss[q], add=True)
        for q in range(RING):
            if dss[q] is not None:
                dss[q].wait()
        return 0

    lax.fori_loop(0, NBLK // 2, body2, 0)
    plsc.subcore_barrier()

    obase = c * HALFP + s * RPT
    pltpu.sync_copy(acc.at[pl.ds(zbase, RPT)], out.at[pl.ds(obase, RPT)])


_sc_layer = functools.partial(
    pl.kernel,
    out_type=jax.ShapeDtypeStruct((NP, EMB), _f32),
    mesh=_mesh,
    compiler_params=_sc_params,
    scratch_types=[
        pltpu.VMEM((BLK_R, 128), _i32),
        pltpu.VMEM((BLK_E,), _i32),
        pltpu.VMEM((BLK_E,), _f32),
        pltpu.VMEM((BLK_R, 128), _i32),
        pltpu.VMEM((BLK_E,), _i32),
        pltpu.VMEM((BLK_E,), _f32),
        pltpu.VMEM((RING, 128), _i32),
        pltpu.VMEM((RING * 128, EMB), _f32),
        pltpu.VMEM_SHARED((HALFP, EMB), _f32),
    ] + [pltpu.SemaphoreType.DMA] * 10,
)(_layer_body)


GIDX = 12288
GROWS = 96


def _gather4_body(t0, t1, t2, t3, idx2d, out,
                  idxv, b0, b1, b2, b3, sem):
    c = lax.axis_index("c")
    s = lax.axis_index("s")
    w = s * 2 + c
    for r in range(3):
        gr = w * 3 + r
        pltpu.sync_copy(idx2d.at[gr], idxv)
        gd = [pltpu.async_copy(tb.at[idxv], bb, sem)
              for tb, bb in ((t0, b0), (t1, b1), (t2, b2), (t3, b3))]
        for d in gd:
            d.wait()

        def sm(e, _):
            a = b0[e, pl.ds(0, 16)] + b1[e, pl.ds(0, 16)]
            bsum = b2[e, pl.ds(0, 16)] + b3[e, pl.ds(0, 16)]
            b0[e, pl.ds(0, 16)] = a + bsum
            a = b0[e, pl.ds(16, 16)] + b1[e, pl.ds(16, 16)]
            bsum = b2[e, pl.ds(16, 16)] + b3[e, pl.ds(16, 16)]
            b0[e, pl.ds(16, 16)] = a + bsum
            return 0
        lax.fori_loop(0, 128, sm, 0, unroll=4)
        pltpu.sync_copy(b0, out.at[pl.ds(gr * 128, 128)])


_sc_gather4 = functools.partial(
    pl.kernel,
    out_type=jax.ShapeDtypeStruct((GIDX, EMB), _f32),
    mesh=_mesh,
    compiler_params=_sc_params,
    scratch_types=[
        pltpu.VMEM((128,), _i32),
        pltpu.VMEM((128, EMB), _f32),
        pltpu.VMEM((128, EMB), _f32),
        pltpu.VMEM((128, EMB), _f32),
        pltpu.VMEM((128, EMB), _f32),
        pltpu.SemaphoreType.DMA,
    ],
)(_gather4_body)


def _tc_body(d_ref, a_ref, gt_ref, gi_ref, mem_ref, item_ref, mask_ref,
             w1u_ref, w1i_ref, b1_ref, w2_ref, b2_ref,
             pw1_ref, pb1_ref, pw2_ref, pb2_ref, out_ref):
    f32 = _f32

    def mm(x, y):
        return jax.lax.dot(x, y, preferred_element_type=f32)

    g0 = gt_ref[...]
    dmat = d_ref[...]
    amat = a_ref[...]
    g1 = mm(dmat, mm(amat, g0))
    g2 = mm(dmat, mm(amat, g1))
    g3 = mm(dmat, mm(amat, g2))
    gacc = g0 + g1 + g2 + g3

    cols = lax.broadcasted_iota(_i32, (BSZ, NGROUPS), 1)
    oh = (cols == gi_ref[...]).astype(f32)
    g_pure = mm(oh, gacc)

    item = item_ref[...]
    q = mm(item, w1i_ref[...]) + b1_ref[...]
    w1u = w1u_ref[...]
    w2 = w2_ref[...]
    b2v = b2_ref[...]
    scores = []
    for l in range(MAXLEN):
        m_l = mem_ref[pl.ds(l * BSZ, BSZ), :]
        h = jnp.maximum(mm(m_l, w1u) + q, 0.0)
        scores.append(mm(h, w2) + b2v)
    smat = jnp.concatenate(scores, axis=1)
    smat = jnp.where(mask_ref[...] != 0, jnp.float32(-1e30), smat)
    mx = jnp.max(smat, axis=1, keepdims=True)
    ex = jnp.exp(smat - mx)
    wt = ex / jnp.sum(ex, axis=1, keepdims=True)

    g_att = jnp.zeros((BSZ, EMB), f32)
    for l in range(MAXLEN):
        m_l = mem_ref[pl.ds(l * BSZ, BSZ), :]
        g_att = g_att + m_l * wt[:, l:l + 1]

    group_emb = g_att + g_pure
    elem = group_emb * item
    h2 = (mm(elem, pw1_ref[pl.ds(0, EMB), :])
          + mm(group_emb, pw1_ref[pl.ds(EMB, EMB), :])
          + mm(item, pw1_ref[pl.ds(2 * EMB, EMB), :])
          + pb1_ref[...])
    h2 = jnp.maximum(h2, 0.0)
    z = mm(h2, pw2_ref[...]) + pb2_ref[...]
    out_ref[...] = 1.0 / (1.0 + jnp.exp(-z))


def _tc_main(D, A, gt, gi, mem, item, mask,
             w1u, w1i, b1, w2, b2, pw1, pb1, pw2, pb2):
    return pl.pallas_call(
        _tc_body,
        out_shape=jax.ShapeDtypeStruct((BSZ, 1), _f32),
    )(D, A, gt, gi, mem, item, mask, w1u, w1i, b1, w2, b2,
      pw1, pb1, pw2, pb2)


def _remap(r):
    return r + jnp.where(r >= NUSERS, 48, 0).astype(_i32)


def kernel(user_table, item_table, group_table, adj_idx, adj_val, D, A,
           attn_w1, attn_b1, attn_w2, attn_b2,
           pred_w1, pred_b1, pred_w2, pred_b2,
           group_inputs, item_inputs, member_masked, mask):
    pad = PADJ - NNZ
    row_p = jnp.pad(_remap(adj_idx[0]), (0, pad))
    col_p = jnp.pad(_remap(adj_idx[1]), (0, pad)).reshape(EROWS, 128)
    val_p = jnp.pad(adj_val, (0, pad))

    zpad = jnp.zeros((48, EMB), _f32)
    ui_p = jnp.concatenate([user_table, zpad, item_table, zpad], axis=0)

    e1 = _sc_layer(col_p, row_p, val_p, ui_p)
    e2 = _sc_layer(col_p, row_p, val_p, e1)
    e3 = _sc_layer(col_p, row_p, val_p, e2)

    gidx = jnp.concatenate([
        member_masked.T.reshape(-1),
        item_inputs + HALFP,
        jnp.zeros((GIDX - BSZ * MAXLEN - BSZ,), _i32),
    ]).reshape(GROWS, 128)
    gsum = _sc_gather4(ui_p, e1, e2, e3, gidx)
    mem = gsum[:BSZ * MAXLEN]
    item_emb = gsum[BSZ * MAXLEN:BSZ * MAXLEN + BSZ]

    return _tc_main(
        D, A, group_table, group_inputs.reshape(BSZ, 1), mem, item_emb,
        mask, attn_w1[:EMB], attn_w1[EMB:], attn_b1.reshape(1, 1 * EMB),
        attn_w2, attn_b2.reshape(1, 1), pred_w1, pred_b1.reshape(1, 8),
        pred_w2, pred_b2.reshape(1, 1))


# speedup vs baseline: 6.6859x; 1.0014x over previous
import functools

import jax
import jax.numpy as jnp
from jax import lax
from jax.experimental import pallas as pl
from jax.experimental.pallas import tpu as pltpu
from jax.experimental.pallas import tpu_sc as plsc

EMB = 32
NUSERS = 50000
NGROUPS = 2048
BSZ = 1024
MAXLEN = 10
NNZ = 1600000
PADJ = 1638400
EROWS = 12800
HALFP = 50048
NP = 2 * HALFP
TILE_EROWS = 800
BLK_R = 8
BLK_E = BLK_R * 128
NBLK = TILE_EROWS // BLK_R
NSUB = BLK_R
RING = 4
RPT = HALFP // 16

_mesh = plsc.VectorSubcoreMesh(core_axis_name="c", subcore_axis_name="s")
_sc_params = pltpu.CompilerParams(use_tc_tiling_on_sc=False)

_f32 = jnp.float32
_i32 = jnp.int32


def _layer_body(col2d, rowf, valf, emb, out,
                sc0, sr0, sv0, sc1, sr1, sv1, lidx, gb, acc,
                si0, si1, sg0, sg1, sg2, sg3, ss0, ss1, ss2, ss3):
    c = lax.axis_index("c")
    s = lax.axis_index("s")
    GBR = RING * 128

    def zb(e, _):
        gb[e, pl.ds(0, 16)] = jnp.zeros((16,), _f32)
        gb[e, pl.ds(16, 16)] = jnp.zeros((16,), _f32)
        return 0
    lax.fori_loop(0, GBR, zb, 0, unroll=4)
    zbase = s * RPT
    for k in range(RPT // GBR):
        pltpu.sync_copy(gb, acc.at[pl.ds(zbase + k * GBR, GBR)])
    _rem = RPT % GBR
    pltpu.sync_copy(gb.at[pl.ds(0, _rem)],
                    acc.at[pl.ds(zbase + (RPT // GBR) * GBR, _rem)])
    plsc.subcore_barrier()

    base_row = c * HALFP
    tile_er0 = s * TILE_EROWS
    iota16 = lax.broadcasted_iota(_i32, (16,), 0)
    sg = (sg0, sg1, sg2, sg3)
    ss = (ss0, ss1, ss2, ss3)
    stages = ((sc0, sr0, sv0), (sc1, sr1, sv1))

    def wait_scatter(q):
        pltpu.make_async_copy(gb.at[pl.ds(q * 128, 128)],
                              acc.at[lidx.at[q]], ss[q]).wait()

    def make_body(first):
        def body2(g, _):
            rb0 = tile_er0 + g * 2 * BLK_R
            eb0 = rb0 * 128
            dst0 = [pltpu.async_copy(col2d.at[pl.ds(rb0, BLK_R)], sc0, si0),
                    pltpu.async_copy(rowf.at[pl.ds(eb0, BLK_E)], sr0, si0),
                    pltpu.async_copy(valf.at[pl.ds(eb0, BLK_E)], sv0, si0)]
            dst1 = [pltpu.async_copy(
                        col2d.at[pl.ds(rb0 + BLK_R, BLK_R)], sc1, si1),
                    pltpu.async_copy(
                        rowf.at[pl.ds(eb0 + BLK_E, BLK_E)], sr1, si1),
                    pltpu.async_copy(
                        valf.at[pl.ds(eb0 + BLK_E, BLK_E)], sv1, si1)]
            for d in dst0:
                d.wait()

            def issue_gather(u):
                pb, sub = divmod(u, NSUB)
                q = u % RING
                return pltpu.async_copy(emb.at[stages[pb][0].at[sub]],
                                        gb.at[pl.ds(q * 128, 128)], sg[q])

            dsg = [None] * RING
            if not first:
                wait_scatter(0)
            dsg[0] = issue_gather(0)
            if not first:
                wait_scatter(1)
            dsg[1] = issue_gather(1)
            for u in range(2 * NSUB):
                pb, sub = divmod(u, NSUB)
                q = u % RING
                un = u + 2
                if un < 2 * NSUB:
                    if un == NSUB:
                        for d in dst1:
                            d.wait()
                    qn = un % RING
                    if (not first) or un >= RING:
                        wait_scatter(qn)
                    dsg[qn] = issue_gather(un)
                dsg[q].wait()
                _, strow, stval = stages[pb]
                vo = sub * 128
                for k in range(8):
                    off = vo + k * 16
                    rv = strow[pl.ds(off, 16)]
                    lr = rv - base_row
                    ok = (lr >= 0) & (lr < HALFP)
                    spread = zbase + off + iota16
                    lidx[q, pl.ds(k * 16, 16)] = jnp.where(ok, lr, spread)
                    stval[pl.ds(off, 16)] = jnp.where(
                        ok, stval[pl.ds(off, 16)], 0.0)

                def mul16(m, _, q=q, stval=stval, vo=vo):
                    vv = stval[pl.ds(vo + m * 16, 16)]
                    for j in range(16):
                        e = q * 128 + m * 16 + j
                        v = vv[j]
                        gb[e, pl.ds(0, 16)] = gb[e, pl.ds(0, 16)] * v
                        gb[e, pl.ds(16, 16)] = gb[e, pl.ds(16, 16)] * v
                    return 0
                lax.fori_loop(0, 8, mul16, 0)

                pltpu.async_copy(gb.at[pl.ds(q * 128, 128)],
                                 acc.at[lidx.at[q]], ss[q], add=True)
            return 0
        return body2

    make_body(True)(0, 0)
    lax.fori_loop(1, NBLK // 2, make_body(False), 0)
    for q in range(RING):
        wait_scatter(q)
    plsc.subcore_barrier()

    obase = c * HALFP + s * RPT
    pltpu.sync_copy(acc.at[pl.ds(zbase, RPT)], out.at[pl.ds(obase, RPT)])


_sc_layer = functools.partial(
    pl.kernel,
    out_type=jax.ShapeDtypeStruct((NP, EMB), _f32),
    mesh=_mesh,
    compiler_params=_sc_params,
    scratch_types=[
        pltpu.VMEM((BLK_R, 128), _i32),
        pltpu.VMEM((BLK_E,), _i32),
        pltpu.VMEM((BLK_E,), _f32),
        pltpu.VMEM((BLK_R, 128), _i32),
        pltpu.VMEM((BLK_E,), _i32),
        pltpu.VMEM((BLK_E,), _f32),
        pltpu.VMEM((RING, 128), _i32),
        pltpu.VMEM((RING * 128, EMB), _f32),
        pltpu.VMEM_SHARED((HALFP, EMB), _f32),
    ] + [pltpu.SemaphoreType.DMA] * 10,
)(_layer_body)


GIDX = 12288
GROWS = 96


def _gather4_body(t0, t1, t2, t3, idx2d, out,
                  idxv, b0, b1, b2, b3, sem):
    c = lax.axis_index("c")
    s = lax.axis_index("s")
    w = s * 2 + c
    for r in range(3):
        gr = w * 3 + r
        pltpu.sync_copy(idx2d.at[gr], idxv)
        gd = [pltpu.async_copy(tb.at[idxv], bb, sem)
              for tb, bb in ((t0, b0), (t1, b1), (t2, b2), (t3, b3))]
        for d in gd:
            d.wait()

        def sm(e, _):
            a = b0[e, pl.ds(0, 16)] + b1[e, pl.ds(0, 16)]
            bsum = b2[e, pl.ds(0, 16)] + b3[e, pl.ds(0, 16)]
            b0[e, pl.ds(0, 16)] = a + bsum
            a = b0[e, pl.ds(16, 16)] + b1[e, pl.ds(16, 16)]
            bsum = b2[e, pl.ds(16, 16)] + b3[e, pl.ds(16, 16)]
            b0[e, pl.ds(16, 16)] = a + bsum
            return 0
        lax.fori_loop(0, 128, sm, 0, unroll=4)
        pltpu.sync_copy(b0, out.at[pl.ds(gr * 128, 128)])


_sc_gather4 = functools.partial(
    pl.kernel,
    out_type=jax.ShapeDtypeStruct((GIDX, EMB), _f32),
    mesh=_mesh,
    compiler_params=_sc_params,
    scratch_types=[
        pltpu.VMEM((128,), _i32),
        pltpu.VMEM((128, EMB), _f32),
        pltpu.VMEM((128, EMB), _f32),
        pltpu.VMEM((128, EMB), _f32),
        pltpu.VMEM((128, EMB), _f32),
        pltpu.SemaphoreType.DMA,
    ],
)(_gather4_body)


def _tc_body(d_ref, a_ref, gt_ref, gi_ref, mem_ref, item_ref, mask_ref,
             w1u_ref, w1i_ref, b1_ref, w2_ref, b2_ref,
             pw1_ref, pb1_ref, pw2_ref, pb2_ref, out_ref):
    f32 = _f32

    def mm(x, y):
        return jax.lax.dot(x, y, preferred_element_type=f32)

    g0 = gt_ref[...]
    dmat = d_ref[...]
    amat = a_ref[...]
    g1 = mm(dmat, mm(amat, g0))
    g2 = mm(dmat, mm(amat, g1))
    g3 = mm(dmat, mm(amat, g2))
    gacc = g0 + g1 + g2 + g3

    cols = lax.broadcasted_iota(_i32, (BSZ, NGROUPS), 1)
    oh = (cols == gi_ref[...]).astype(f32)
    g_pure = mm(oh, gacc)

    item = item_ref[...]
    q = mm(item, w1i_ref[...]) + b1_ref[...]
    w1u = w1u_ref[...]
    w2 = w2_ref[...]
    b2v = b2_ref[...]
    scores = []
    for l in range(MAXLEN):
        m_l = mem_ref[pl.ds(l * BSZ, BSZ), :]
        h = jnp.maximum(mm(m_l, w1u) + q, 0.0)
        scores.append(mm(h, w2) + b2v)
    smat = jnp.concatenate(scores, axis=1)
    smat = jnp.where(mask_ref[...] != 0, jnp.float32(-1e30), smat)
    mx = jnp.max(smat, axis=1, keepdims=True)
    ex = jnp.exp(smat - mx)
    wt = ex / jnp.sum(ex, axis=1, keepdims=True)

    g_att = jnp.zeros((BSZ, EMB), f32)
    for l in range(MAXLEN):
        m_l = mem_ref[pl.ds(l * BSZ, BSZ), :]
        g_att = g_att + m_l * wt[:, l:l + 1]

    group_emb = g_att + g_pure
    elem = group_emb * item
    h2 = (mm(elem, pw1_ref[pl.ds(0, EMB), :])
          + mm(group_emb, pw1_ref[pl.ds(EMB, EMB), :])
          + mm(item, pw1_ref[pl.ds(2 * EMB, EMB), :])
          + pb1_ref[...])
    h2 = jnp.maximum(h2, 0.0)
    z = mm(h2, pw2_ref[...]) + pb2_ref[...]
    out_ref[...] = 1.0 / (1.0 + jnp.exp(-z))


def _tc_main(D, A, gt, gi, mem, item, mask,
             w1u, w1i, b1, w2, b2, pw1, pb1, pw2, pb2):
    return pl.pallas_call(
        _tc_body,
        out_shape=jax.ShapeDtypeStruct((BSZ, 1), _f32),
    )(D, A, gt, gi, mem, item, mask, w1u, w1i, b1, w2, b2,
      pw1, pb1, pw2, pb2)


def _remap(r):
    return r + jnp.where(r >= NUSERS, 48, 0).astype(_i32)


def kernel(user_table, item_table, group_table, adj_idx, adj_val, D, A,
           attn_w1, attn_b1, attn_w2, attn_b2,
           pred_w1, pred_b1, pred_w2, pred_b2,
           group_inputs, item_inputs, member_masked, mask):
    pad = PADJ - NNZ
    row_p = jnp.pad(_remap(adj_idx[0]), (0, pad))
    col_p = jnp.pad(_remap(adj_idx[1]), (0, pad)).reshape(EROWS, 128)
    val_p = jnp.pad(adj_val, (0, pad))

    zpad = jnp.zeros((48, EMB), _f32)
    ui_p = jnp.concatenate([user_table, zpad, item_table, zpad], axis=0)

    e1 = _sc_layer(col_p, row_p, val_p, ui_p)
    e2 = _sc_layer(col_p, row_p, val_p, e1)
    e3 = _sc_layer(col_p, row_p, val_p, e2)

    gidx = jnp.concatenate([
        member_masked.T.reshape(-1),
        item_inputs + HALFP,
        jnp.zeros((GIDX - BSZ * MAXLEN - BSZ,), _i32),
    ]).reshape(GROWS, 128)
    gsum = _sc_gather4(ui_p, e1, e2, e3, gidx)
    mem = gsum[:BSZ * MAXLEN]
    item_emb = gsum[BSZ * MAXLEN:BSZ * MAXLEN + BSZ]

    return _tc_main(
        D, A, group_table, group_inputs.reshape(BSZ, 1), mem, item_emb,
        mask, attn_w1[:EMB], attn_w1[EMB:], attn_b1.reshape(1, 1 * EMB),
        attn_w2, attn_b2.reshape(1, 1), pred_w1, pred_b1.reshape(1, 8),
        pred_w2, pred_b2.reshape(1, 1))


# speedup vs baseline: 7.0279x; 1.0512x over previous
import functools

import jax
import jax.numpy as jnp
from jax import lax
from jax.experimental import pallas as pl
from jax.experimental.pallas import tpu as pltpu
from jax.experimental.pallas import tpu_sc as plsc

EMB = 32
NUSERS = 50000
NGROUPS = 2048
BSZ = 1024
MAXLEN = 10
NNZ = 1600000
PADJ = 1638400
EROWS = 12800
HALFP = 50048
NP = 2 * HALFP
TILE_EROWS = 800
BLK_R = 8
BLK_E = BLK_R * 128
NBLK = TILE_EROWS // BLK_R
NSUB = BLK_R
RING = 4
RPT = HALFP // 16

_mesh = plsc.VectorSubcoreMesh(core_axis_name="c", subcore_axis_name="s")
_sc_params = pltpu.CompilerParams(use_tc_tiling_on_sc=False,
                                  needs_layout_passes=False)

_f32 = jnp.float32
_i32 = jnp.int32


def _layer_body(col2d, rowf, valf, emb, out,
                sc0, sr0, sv0, sc1, sr1, sv1, ccol, crow, cval, gb, acc,
                si0, si1, sg0, sg1, sg2, sg3, ss0, ss1, ss2, ss3):
    c = lax.axis_index("c")
    s = lax.axis_index("s")
    GBR = RING * 128

    def zb(e, _):
        gb[e, pl.ds(0, 16)] = jnp.zeros((16,), _f32)
        gb[e, pl.ds(16, 16)] = jnp.zeros((16,), _f32)
        return 0
    lax.fori_loop(0, GBR, zb, 0, unroll=4)
    zbase = s * RPT
    for k in range(RPT // GBR):
        pltpu.sync_copy(gb, acc.at[pl.ds(zbase + k * GBR, GBR)])
    _rem = RPT % GBR
    pltpu.sync_copy(gb.at[pl.ds(0, _rem)],
                    acc.at[pl.ds(zbase + (RPT // GBR) * GBR, _rem)])
    plsc.subcore_barrier()

    base_row = c * HALFP
    tile_er0 = s * TILE_EROWS
    sg = (sg0, sg1, sg2, sg3)
    ss = (ss0, ss1, ss2, ss3)
    stages = ((sc0, sr0, sv0), (sc1, sr1, sv1))

    def compact_block(stc, strow, stval, w):
        for m in range(64):
            off = (m % 8) * 16
            col16 = stc[m // 8, pl.ds(off, 16)]
            r16 = strow[pl.ds(m * 16, 16)]
            v16 = stval[pl.ds(m * 16, 16)]
            lr = r16 - base_row
            ok = (lr >= 0) & (lr < HALFP)
            cs = plsc.cumsum(ok.astype(_i32))
            idx = jnp.where(ok, w + cs - 1, 0)
            plsc.store_scatter(ccol, [idx], col16, mask=ok)
            plsc.store_scatter(crow, [idx], lr, mask=ok)
            plsc.store_scatter(cval, [idx], v16, mask=ok)
            w = w + cs[15]
        return w

    def drain4():
        dg = [pltpu.async_copy(emb.at[ccol.at[pl.ds(k * 128, 128)]],
                               gb.at[pl.ds(k * 128, 128)], sg[k])
              for k in range(RING)]
        dss = []
        for k in range(RING):
            dg[k].wait()

            def mul16(m, _, k=k):
                vv = cval[pl.ds(k * 128 + m * 16, 16)]
                for j in range(16):
                    e = k * 128 + m * 16 + j
                    v = vv[j]
                    gb[e, pl.ds(0, 16)] = gb[e, pl.ds(0, 16)] * v
                    gb[e, pl.ds(16, 16)] = gb[e, pl.ds(16, 16)] * v
                return 0
            lax.fori_loop(0, 8, mul16, 0)
            dss.append(pltpu.async_copy(
                gb.at[pl.ds(k * 128, 128)],
                acc.at[crow.at[pl.ds(k * 128, 128)]], ss[k], add=True))
        for d in dss:
            d.wait()

    def maybe_drain(w):
        def dbody(w):
            drain4()
            for i in range(64):
                ccol[pl.ds(i * 16, 16)] = ccol[pl.ds(512 + i * 16, 16)]
                crow[pl.ds(i * 16, 16)] = crow[pl.ds(512 + i * 16, 16)]
                cval[pl.ds(i * 16, 16)] = cval[pl.ds(512 + i * 16, 16)]
            return w - 512
        return lax.while_loop(lambda w: w >= 512, dbody, w)

    def pair_body(g, w):
        rb0 = tile_er0 + g * 2 * BLK_R
        eb0 = rb0 * 128
        dst0 = [pltpu.async_copy(col2d.at[pl.ds(rb0, BLK_R)], sc0, si0),
                pltpu.async_copy(rowf.at[pl.ds(eb0, BLK_E)], sr0, si0),
                pltpu.async_copy(valf.at[pl.ds(eb0, BLK_E)], sv0, si0)]
        dst1 = [pltpu.async_copy(
                    col2d.at[pl.ds(rb0 + BLK_R, BLK_R)], sc1, si1),
                pltpu.async_copy(
                    rowf.at[pl.ds(eb0 + BLK_E, BLK_E)], sr1, si1),
                pltpu.async_copy(
                    valf.at[pl.ds(eb0 + BLK_E, BLK_E)], sv1, si1)]
        for d in dst0:
            d.wait()
        w = maybe_drain(compact_block(sc0, sr0, sv0, w))
        for d in dst1:
            d.wait()
        w = maybe_drain(compact_block(sc1, sr1, sv1, w))
        return w

    w = lax.fori_loop(0, NBLK // 2, pair_body, jnp.int32(0))

    zpad_i = jnp.zeros((16,), _i32)
    zpad_r = jnp.full((16,), zbase, _i32)
    zpad_f = jnp.zeros((16,), _f32)
    for i in range(32):
        ccol[pl.ds(w + i * 16, 16)] = zpad_i
        crow[pl.ds(w + i * 16, 16)] = zpad_r
        cval[pl.ds(w + i * 16, 16)] = zpad_f
    drain4()
    plsc.subcore_barrier()

    obase = c * HALFP + s * RPT
    pltpu.sync_copy(acc.at[pl.ds(zbase, RPT)], out.at[pl.ds(obase, RPT)])


_sc_layer = functools.partial(
    pl.kernel,
    out_type=jax.ShapeDtypeStruct((NP, EMB), _f32),
    mesh=_mesh,
    compiler_params=_sc_params,
    scratch_types=[
        pltpu.VMEM((BLK_R, 128), _i32),
        pltpu.VMEM((BLK_E,), _i32),
        pltpu.VMEM((BLK_E,), _f32),
        pltpu.VMEM((BLK_R, 128), _i32),
        pltpu.VMEM((BLK_E,), _i32),
        pltpu.VMEM((BLK_E,), _f32),
        pltpu.VMEM((1664,), _i32),
        pltpu.VMEM((1664,), _i32),
        pltpu.VMEM((1664,), _f32),
        pltpu.VMEM((RING * 128, EMB), _f32),
        pltpu.VMEM_SHARED((HALFP, EMB), _f32),
    ] + [pltpu.SemaphoreType.DMA] * 10,
)(_layer_body)


GIDX = 12288
GROWS = 96


def _gather4_body(t0, t1, t2, t3, idx2d, out,
                  idxv, b0, b1, b2, b3, sem):
    c = lax.axis_index("c")
    s = lax.axis_index("s")
    w = s * 2 + c
    for r in range(3):
        gr = w * 3 + r
        pltpu.sync_copy(idx2d.at[gr], idxv)
        gd = [pltpu.async_copy(tb.at[idxv], bb, sem)
              for tb, bb in ((t0, b0), (t1, b1), (t2, b2), (t3, b3))]
        for d in gd:
            d.wait()

        def sm(e, _):
            a = b0[e, pl.ds(0, 16)] + b1[e, pl.ds(0, 16)]
            bsum = b2[e, pl.ds(0, 16)] + b3[e, pl.ds(0, 16)]
            b0[e, pl.ds(0, 16)] = a + bsum
            a = b0[e, pl.ds(16, 16)] + b1[e, pl.ds(16, 16)]
            bsum = b2[e, pl.ds(16, 16)] + b3[e, pl.ds(16, 16)]
            b0[e, pl.ds(16, 16)] = a + bsum
            return 0
        lax.fori_loop(0, 128, sm, 0, unroll=4)
        pltpu.sync_copy(b0, out.at[pl.ds(gr * 128, 128)])


_sc_gather4 = functools.partial(
    pl.kernel,
    out_type=jax.ShapeDtypeStruct((GIDX, EMB), _f32),
    mesh=_mesh,
    compiler_params=_sc_params,
    scratch_types=[
        pltpu.VMEM((128,), _i32),
        pltpu.VMEM((128, EMB), _f32),
        pltpu.VMEM((128, EMB), _f32),
        pltpu.VMEM((128, EMB), _f32),
        pltpu.VMEM((128, EMB), _f32),
        pltpu.SemaphoreType.DMA,
    ],
)(_gather4_body)


def _tc_body(d_ref, a_ref, gt_ref, gi_ref, mem_ref, item_ref, mask_ref,
             w1u_ref, w1i_ref, b1_ref, w2_ref, b2_ref,
             pw1_ref, pb1_ref, pw2_ref, pb2_ref, out_ref):
    f32 = _f32

    def mm(x, y):
        return jax.lax.dot(x, y, preferred_element_type=f32)

    g0 = gt_ref[...]
    dmat = d_ref[...]
    amat = a_ref[...]
    g1 = mm(dmat, mm(amat, g0))
    g2 = mm(dmat, mm(amat, g1))
    g3 = mm(dmat, mm(amat, g2))
    gacc = g0 + g1 + g2 + g3

    cols = lax.broadcasted_iota(_i32, (BSZ, NGROUPS), 1)
    oh = (cols == gi_ref[...]).astype(f32)
    g_pure = mm(oh, gacc)

    item = item_ref[...]
    q = mm(item, w1i_ref[...]) + b1_ref[...]
    w1u = w1u_ref[...]
    w2 = w2_ref[...]
    b2v = b2_ref[...]
    scores = []
    for l in range(MAXLEN):
        m_l = mem_ref[pl.ds(l * BSZ, BSZ), :]
        h = jnp.maximum(mm(m_l, w1u) + q, 0.0)
        scores.append(mm(h, w2) + b2v)
    smat = jnp.concatenate(scores, axis=1)
    smat = jnp.where(mask_ref[...] != 0, jnp.float32(-1e30), smat)
    mx = jnp.max(smat, axis=1, keepdims=True)
    ex = jnp.exp(smat - mx)
    wt = ex / jnp.sum(ex, axis=1, keepdims=True)

    g_att = jnp.zeros((BSZ, EMB), f32)
    for l in range(MAXLEN):
        m_l = mem_ref[pl.ds(l * BSZ, BSZ), :]
        g_att = g_att + m_l * wt[:, l:l + 1]

    group_emb = g_att + g_pure
    elem = group_emb * item
    h2 = (mm(elem, pw1_ref[pl.ds(0, EMB), :])
          + mm(group_emb, pw1_ref[pl.ds(EMB, EMB), :])
          + mm(item, pw1_ref[pl.ds(2 * EMB, EMB), :])
          + pb1_ref[...])
    h2 = jnp.maximum(h2, 0.0)
    z = mm(h2, pw2_ref[...]) + pb2_ref[...]
    out_ref[...] = 1.0 / (1.0 + jnp.exp(-z))


def _tc_main(D, A, gt, gi, mem, item, mask,
             w1u, w1i, b1, w2, b2, pw1, pb1, pw2, pb2):
    return pl.pallas_call(
        _tc_body,
        out_shape=jax.ShapeDtypeStruct((BSZ, 1), _f32),
    )(D, A, gt, gi, mem, item, mask, w1u, w1i, b1, w2, b2,
      pw1, pb1, pw2, pb2)


def _remap(r):
    return r + jnp.where(r >= NUSERS, 48, 0).astype(_i32)


def kernel(user_table, item_table, group_table, adj_idx, adj_val, D, A,
           attn_w1, attn_b1, attn_w2, attn_b2,
           pred_w1, pred_b1, pred_w2, pred_b2,
           group_inputs, item_inputs, member_masked, mask):
    pad = PADJ - NNZ
    row_p = jnp.pad(_remap(adj_idx[0]), (0, pad))
    col_p = jnp.pad(_remap(adj_idx[1]), (0, pad)).reshape(EROWS, 128)
    val_p = jnp.pad(adj_val, (0, pad))

    zpad = jnp.zeros((48, EMB), _f32)
    ui_p = jnp.concatenate([user_table, zpad, item_table, zpad], axis=0)

    e1 = _sc_layer(col_p, row_p, val_p, ui_p)
    e2 = _sc_layer(col_p, row_p, val_p, e1)
    e3 = _sc_layer(col_p, row_p, val_p, e2)

    gidx = jnp.concatenate([
        member_masked.T.reshape(-1),
        item_inputs + HALFP,
        jnp.zeros((GIDX - BSZ * MAXLEN - BSZ,), _i32),
    ]).reshape(GROWS, 128)
    gsum = _sc_gather4(ui_p, e1, e2, e3, gidx)
    mem = gsum[:BSZ * MAXLEN]
    item_emb = gsum[BSZ * MAXLEN:BSZ * MAXLEN + BSZ]

    return _tc_main(
        D, A, group_table, group_inputs.reshape(BSZ, 1), mem, item_emb,
        mask, attn_w1[:EMB], attn_w1[EMB:], attn_b1.reshape(1, 1 * EMB),
        attn_w2, attn_b2.reshape(1, 1), pred_w1, pred_b1.reshape(1, 8),
        pred_w2, pred_b2.reshape(1, 1))
